# baseline (speedup 1.0000x reference)
import jax
import jax.numpy as jnp
from jax.experimental import pallas as pl

_NN = 254
_NP = 256
_H = 128
_BB = 4

_BF = jnp.bfloat16


def _tree_kernel(x_ref, e_ref, rel_ref, hp_ref, t_ref,
                 Wt_ref, bt_ref, Wmsg_ref, bmsg_ref,
                 Wih_ref, Whh_ref, bih_ref, bhh_ref,
                 Wp1_ref, bp1_ref, Wp2_ref, bp2_ref,
                 out_ref):
    f32 = jnp.float32
    t = t_ref[0, 0]

    half = _H // 2
    kf = jax.lax.broadcasted_iota(jnp.int32, (1, half), 1).astype(f32)
    freqs = jnp.exp(-jnp.log(10000.0) * kf / half)
    ang = t * freqs
    emb = jnp.concatenate([jnp.sin(ang), jnp.cos(ang)], axis=1)
    temb = jnp.dot(emb, Wt_ref[...], preferred_element_type=f32) + bt_ref[...]

    Wtop = Wmsg_ref[:_H, :]
    Wbot = Wmsg_ref[_H:, :]
    Wd = Wtop - Wbot
    iota_l = jax.lax.broadcasted_iota(jnp.int32, (_NP, _NP), 1)
    iota_s = jax.lax.broadcasted_iota(jnp.int32, (_NP, _NN - 2), 0)

    zrows = jnp.zeros((_NP - _NN, _H), _BF)
    x_all = jnp.concatenate(
        [p for b in range(_BB) for p in (x_ref[b], zrows)], axis=0)

    u_all = jnp.dot(x_all, Wd, preferred_element_type=f32) + bmsg_ref[...]
    v_all = jnp.dot(x_all, Wbot, preferred_element_type=f32).astype(_BF)

    gnn_parts = []
    h0_parts = []
    ponehots = []
    for b in range(_BB):
        e = e_ref[b]
        u = u_all[b * _NP:(b + 1) * _NP]
        v = v_all[b * _NP:(b + 1) * _NP]
        gnn = jnp.zeros((_NP, _H), f32)
        for k in range(3):
            idx_col = e[:, k:k + 1]
            onehot = (iota_l == idx_col).astype(_BF)
            if k == 0:
                ponehots.append(onehot)
            nv = jnp.dot(onehot, v, preferred_element_type=f32)
            gnn = gnn + jnp.maximum(u + nv, 0.0)
        gnn_parts.append(gnn.astype(_BF))

        rel = rel_ref[b]
        scat = (iota_s == rel).astype(_BF)
        h0_parts.append(jnp.dot(scat, hp_ref[b],
                                preferred_element_type=f32))

    gnn_all = jnp.concatenate(gnn_parts, axis=0)
    h0_all = jnp.concatenate(h0_parts, axis=0)

    gi = jax.lax.dot_general(gnn_all, Wih_ref[...], (((1,), (1,)), ((), ())),
                             preferred_element_type=f32) + bih_ref[...]
    gh = jax.lax.dot_general(h0_all.astype(_BF), Whh_ref[...],
                             (((1,), (1,)), ((), ())),
                             preferred_element_type=f32) + bhh_ref[...]
    r = jax.nn.sigmoid(gi[:, :_H] + gh[:, :_H])
    z = jax.nn.sigmoid(gi[:, _H:2 * _H] + gh[:, _H:2 * _H])
    n = jnp.tanh(gi[:, 2 * _H:] + r * gh[:, 2 * _H:])
    h_new = (1.0 - z) * n + z * h0_all

    h_bf = h_new.astype(_BF)
    ef_parts = []
    for b in range(_BB):
        hb = h_bf[b * _NP:(b + 1) * _NP]
        pf = jnp.dot(ponehots[b], hb, preferred_element_type=f32)
        ef_parts.append(jnp.maximum(h_new[b * _NP:(b + 1) * _NP], pf))
    ef_all = jnp.concatenate(ef_parts, axis=0)

    ee = jnp.maximum(
        jnp.dot(ef_all.astype(_BF), Wp1_ref[...], preferred_element_type=f32)
        + bp1_ref[...] + temb, 0.0)
    o = jnp.dot(ee.astype(_BF), Wp2_ref[...],
                preferred_element_type=f32) + bp2_ref[0, 0]
    for b in range(_BB):
        out_ref[b] = o[b * _NP:b * _NP + (_NN - 1)]


def kernel(node_features, edge_index, rel_pos, h_prev, t, W_t, b_t, W_msg,
           b_msg, W_ih, W_hh, b_ih, b_hh, W_p1, b_p1, W_p2, b_p2):
    B, NN, NT = node_features.shape
    H = W_t.shape[0]
    NP = _NP

    e_pad = jnp.pad(edge_index, ((0, 0), (0, NP - NN), (0, 0)))
    x_bf = node_features.astype(_BF)
    rel3 = rel_pos.reshape(B, 1, NN - 2)
    hp3 = h_prev.reshape(B, NN - 2, H).astype(_BF)
    t2 = t.reshape(1, 1)
    b_t2 = b_t.reshape(1, H)
    b_msg2 = b_msg.reshape(1, H)
    b_ih2 = b_ih.reshape(1, 3 * H)
    b_hh2 = b_hh.reshape(1, 3 * H)
    b_p12 = b_p1.reshape(1, H)
    b_p22 = b_p2.reshape(1, 1)

    def bmap(b):
        return (b, 0, 0)

    def wmap2(b):
        return (0, 0)

    out = pl.pallas_call(
        _tree_kernel,
        grid=(B // _BB,),
        in_specs=[
            pl.BlockSpec((_BB, NN, NT), bmap),
            pl.BlockSpec((_BB, NP, 3), bmap),
            pl.BlockSpec((_BB, 1, NN - 2), bmap),
            pl.BlockSpec((_BB, NN - 2, H), bmap),
            pl.BlockSpec((1, 1), wmap2),
            pl.BlockSpec((H, H), wmap2),
            pl.BlockSpec((1, H), wmap2),
            pl.BlockSpec((2 * NT, H), wmap2),
            pl.BlockSpec((1, H), wmap2),
            pl.BlockSpec((3 * H, H), wmap2),
            pl.BlockSpec((3 * H, H), wmap2),
            pl.BlockSpec((1, 3 * H), wmap2),
            pl.BlockSpec((1, 3 * H), wmap2),
            pl.BlockSpec((H, H), wmap2),
            pl.BlockSpec((1, H), wmap2),
            pl.BlockSpec((H, 1), wmap2),
            pl.BlockSpec((1, 1), wmap2),
        ],
        out_specs=pl.BlockSpec((_BB, NN - 1, 1), bmap),
        out_shape=jax.ShapeDtypeStruct((B, NN - 1, 1), jnp.float32),
    )(x_bf, e_pad, rel3, hp3, t2, W_t, b_t2, W_msg.astype(_BF), b_msg2,
      W_ih.astype(_BF), W_hh.astype(_BF), b_ih2, b_hh2,
      W_p1.astype(_BF), b_p12, W_p2.astype(_BF), b_p22)
    return out.reshape(B, NN - 1)


# speedup vs baseline: 1.2903x; 1.2903x over previous
import jax
import jax.numpy as jnp
from jax.experimental import pallas as pl

_NN = 254
_NP = 256
_H = 128
_BB = 8


def _tree_kernel(x_ref, e_ref, rel_ref, hp_ref, t_ref,
                 Wt_ref, bt_ref, Wmsg_ref, bmsg_ref,
                 Wih_ref, Whh_ref, bih_ref, bhh_ref,
                 Wp1_ref, bp1_ref, Wp2_ref, bp2_ref,
                 out_ref):
    f32 = jnp.float32
    t = t_ref[0, 0]

    half = _H // 2
    kf = jax.lax.broadcasted_iota(jnp.int32, (1, half), 1).astype(f32)
    freqs = jnp.exp(-jnp.log(10000.0) * kf / half)
    ang = t * freqs
    emb = jnp.concatenate([jnp.sin(ang), jnp.cos(ang)], axis=1)
    temb = jnp.dot(emb, Wt_ref[...], preferred_element_type=f32) + bt_ref[...]

    Wtop = Wmsg_ref[:_H, :]
    Wbot = Wmsg_ref[_H:, :]
    Wd = Wtop - Wbot
    iota_l = jax.lax.broadcasted_iota(jnp.int32, (_NP, _NP), 1)
    iota_s = jax.lax.broadcasted_iota(jnp.int32, (_NP, _NN - 2), 0)

    zrows = jnp.zeros((_NP - _NN, _H), f32)
    x_all = jnp.concatenate(
        [p for b in range(_BB) for p in (x_ref[b], zrows)], axis=0)

    u_all = jnp.dot(x_all, Wd, preferred_element_type=f32) + bmsg_ref[...]
    v_all = jnp.dot(x_all, Wbot, preferred_element_type=f32)

    gnn_parts = []
    h0_parts = []
    ponehots = []
    for b in range(_BB):
        e = e_ref[b]
        u = u_all[b * _NP:(b + 1) * _NP]
        v = v_all[b * _NP:(b + 1) * _NP]
        gnn = jnp.zeros((_NP, _H), f32)
        for k in range(3):
            idx_col = e[:, k:k + 1]
            onehot = (iota_l == idx_col).astype(f32)
            if k == 0:
                ponehots.append(onehot)
            nv = jnp.dot(onehot, v, preferred_element_type=f32)
            gnn = gnn + jnp.maximum(u + nv, 0.0)
        gnn_parts.append(gnn)

        rel = rel_ref[b]
        scat = (iota_s == rel).astype(f32)
        h0_parts.append(jnp.dot(scat, hp_ref[b],
                                preferred_element_type=f32))

    gnn_all = jnp.concatenate(gnn_parts, axis=0)
    h0_all = jnp.concatenate(h0_parts, axis=0)

    gi = jax.lax.dot_general(gnn_all, Wih_ref[...], (((1,), (1,)), ((), ())),
                             preferred_element_type=f32) + bih_ref[...]
    gh = jax.lax.dot_general(h0_all, Whh_ref[...], (((1,), (1,)), ((), ())),
                             preferred_element_type=f32) + bhh_ref[...]
    r = jax.nn.sigmoid(gi[:, :_H] + gh[:, :_H])
    z = jax.nn.sigmoid(gi[:, _H:2 * _H] + gh[:, _H:2 * _H])
    n = jnp.tanh(gi[:, 2 * _H:] + r * gh[:, 2 * _H:])
    h_new = (1.0 - z) * n + z * h0_all

    ef_parts = []
    for b in range(_BB):
        hb = h_new[b * _NP:(b + 1) * _NP]
        pf = jnp.dot(ponehots[b], hb, preferred_element_type=f32)
        ef_parts.append(jnp.maximum(hb, pf))
    ef_all = jnp.concatenate(ef_parts, axis=0)

    ee = jnp.maximum(
        jnp.dot(ef_all, Wp1_ref[...], preferred_element_type=f32)
        + bp1_ref[...] + temb, 0.0)
    o = jnp.dot(ee, Wp2_ref[...], preferred_element_type=f32) + bp2_ref[0, 0]
    for b in range(_BB):
        out_ref[b] = o[b * _NP:b * _NP + (_NN - 1)]


def kernel(node_features, edge_index, rel_pos, h_prev, t, W_t, b_t, W_msg,
           b_msg, W_ih, W_hh, b_ih, b_hh, W_p1, b_p1, W_p2, b_p2):
    B, NN, NT = node_features.shape
    H = W_t.shape[0]
    NP = _NP

    e_pad = jnp.pad(edge_index, ((0, 0), (0, NP - NN), (0, 0)))
    rel3 = rel_pos.reshape(B, 1, NN - 2)
    hp3 = h_prev.reshape(B, NN - 2, H)
    t2 = t.reshape(1, 1)
    b_t2 = b_t.reshape(1, H)
    b_msg2 = b_msg.reshape(1, H)
    b_ih2 = b_ih.reshape(1, 3 * H)
    b_hh2 = b_hh.reshape(1, 3 * H)
    b_p12 = b_p1.reshape(1, H)
    b_p22 = b_p2.reshape(1, 1)

    def bmap(b):
        return (b, 0, 0)

    def wmap2(b):
        return (0, 0)

    out = pl.pallas_call(
        _tree_kernel,
        grid=(B // _BB,),
        in_specs=[
            pl.BlockSpec((_BB, NN, NT), bmap),
            pl.BlockSpec((_BB, NP, 3), bmap),
            pl.BlockSpec((_BB, 1, NN - 2), bmap),
            pl.BlockSpec((_BB, NN - 2, H), bmap),
            pl.BlockSpec((1, 1), wmap2),
            pl.BlockSpec((H, H), wmap2),
            pl.BlockSpec((1, H), wmap2),
            pl.BlockSpec((2 * NT, H), wmap2),
            pl.BlockSpec((1, H), wmap2),
            pl.BlockSpec((3 * H, H), wmap2),
            pl.BlockSpec((3 * H, H), wmap2),
            pl.BlockSpec((1, 3 * H), wmap2),
            pl.BlockSpec((1, 3 * H), wmap2),
            pl.BlockSpec((H, H), wmap2),
            pl.BlockSpec((1, H), wmap2),
            pl.BlockSpec((H, 1), wmap2),
            pl.BlockSpec((1, 1), wmap2),
        ],
        out_specs=pl.BlockSpec((_BB, NN - 1, 1), bmap),
        out_shape=jax.ShapeDtypeStruct((B, NN - 1, 1), jnp.float32),
    )(node_features, e_pad, rel3, hp3, t2, W_t, b_t2, W_msg, b_msg2,
      W_ih, W_hh, b_ih2, b_hh2, W_p1, b_p12, W_p2, b_p22)
    return out.reshape(B, NN - 1)


# speedup vs baseline: 1.3097x; 1.0150x over previous
import jax
import jax.numpy as jnp
from jax.experimental import pallas as pl

_NN = 254
_NP = 256
_H = 128
_BB = 16


def _tree_kernel(x_ref, e_ref, rel_ref, hp_ref, t_ref,
                 Wt_ref, bt_ref, Wmsg_ref, bmsg_ref,
                 Wih_ref, Whh_ref, bih_ref, bhh_ref,
                 Wp1_ref, bp1_ref, Wp2_ref, bp2_ref,
                 out_ref):
    f32 = jnp.float32
    t = t_ref[0, 0]

    half = _H // 2
    kf = jax.lax.broadcasted_iota(jnp.int32, (1, half), 1).astype(f32)
    freqs = jnp.exp(-jnp.log(10000.0) * kf / half)
    ang = t * freqs
    emb = jnp.concatenate([jnp.sin(ang), jnp.cos(ang)], axis=1)
    temb = jnp.dot(emb, Wt_ref[...], preferred_element_type=f32) + bt_ref[...]

    Wtop = Wmsg_ref[:_H, :]
    Wbot = Wmsg_ref[_H:, :]
    Wd = Wtop - Wbot
    iota_l = jax.lax.broadcasted_iota(jnp.int32, (_NP, _NP), 1)
    iota_s = jax.lax.broadcasted_iota(jnp.int32, (_NP, _NN - 2), 0)

    zrows = jnp.zeros((_NP - _NN, _H), f32)
    x_all = jnp.concatenate(
        [p for b in range(_BB) for p in (x_ref[b], zrows)], axis=0)

    u_all = jnp.dot(x_all, Wd, preferred_element_type=f32) + bmsg_ref[...]
    v_all = jnp.dot(x_all, Wbot, preferred_element_type=f32)

    gnn_parts = []
    h0_parts = []
    ponehots = []
    for b in range(_BB):
        e = e_ref[b]
        u = u_all[b * _NP:(b + 1) * _NP]
        v = v_all[b * _NP:(b + 1) * _NP]
        gnn = jnp.zeros((_NP, _H), f32)
        for k in range(3):
            idx_col = e[:, k:k + 1]
            onehot = (iota_l == idx_col).astype(f32)
            if k == 0:
                ponehots.append(onehot)
            nv = jnp.dot(onehot, v, preferred_element_type=f32)
            gnn = gnn + jnp.maximum(u + nv, 0.0)
        gnn_parts.append(gnn)

        rel = rel_ref[b]
        scat = (iota_s == rel).astype(f32)
        h0_parts.append(jnp.dot(scat, hp_ref[b],
                                preferred_element_type=f32))

    gnn_all = jnp.concatenate(gnn_parts, axis=0)
    h0_all = jnp.concatenate(h0_parts, axis=0)

    gi = jax.lax.dot_general(gnn_all, Wih_ref[...], (((1,), (1,)), ((), ())),
                             preferred_element_type=f32) + bih_ref[...]
    gh = jax.lax.dot_general(h0_all, Whh_ref[...], (((1,), (1,)), ((), ())),
                             preferred_element_type=f32) + bhh_ref[...]
    r = jax.nn.sigmoid(gi[:, :_H] + gh[:, :_H])
    z = jax.nn.sigmoid(gi[:, _H:2 * _H] + gh[:, _H:2 * _H])
    n = jnp.tanh(gi[:, 2 * _H:] + r * gh[:, 2 * _H:])
    h_new = (1.0 - z) * n + z * h0_all

    ef_parts = []
    for b in range(_BB):
        hb = h_new[b * _NP:(b + 1) * _NP]
        pf = jnp.dot(ponehots[b], hb, preferred_element_type=f32)
        ef_parts.append(jnp.maximum(hb, pf))
    ef_all = jnp.concatenate(ef_parts, axis=0)

    ee = jnp.maximum(
        jnp.dot(ef_all, Wp1_ref[...], preferred_element_type=f32)
        + bp1_ref[...] + temb, 0.0)
    o = jnp.dot(ee, Wp2_ref[...], preferred_element_type=f32) + bp2_ref[0, 0]
    for b in range(_BB):
        out_ref[b] = o[b * _NP:b * _NP + (_NN - 1)]


def kernel(node_features, edge_index, rel_pos, h_prev, t, W_t, b_t, W_msg,
           b_msg, W_ih, W_hh, b_ih, b_hh, W_p1, b_p1, W_p2, b_p2):
    B, NN, NT = node_features.shape
    H = W_t.shape[0]
    NP = _NP

    e_pad = jnp.pad(edge_index, ((0, 0), (0, NP - NN), (0, 0)))
    rel3 = rel_pos.reshape(B, 1, NN - 2)
    hp3 = h_prev.reshape(B, NN - 2, H)
    t2 = t.reshape(1, 1)
    b_t2 = b_t.reshape(1, H)
    b_msg2 = b_msg.reshape(1, H)
    b_ih2 = b_ih.reshape(1, 3 * H)
    b_hh2 = b_hh.reshape(1, 3 * H)
    b_p12 = b_p1.reshape(1, H)
    b_p22 = b_p2.reshape(1, 1)

    def bmap(b):
        return (b, 0, 0)

    def wmap2(b):
        return (0, 0)

    out = pl.pallas_call(
        _tree_kernel,
        grid=(B // _BB,),
        in_specs=[
            pl.BlockSpec((_BB, NN, NT), bmap),
            pl.BlockSpec((_BB, NP, 3), bmap),
            pl.BlockSpec((_BB, 1, NN - 2), bmap),
            pl.BlockSpec((_BB, NN - 2, H), bmap),
            pl.BlockSpec((1, 1), wmap2),
            pl.BlockSpec((H, H), wmap2),
            pl.BlockSpec((1, H), wmap2),
            pl.BlockSpec((2 * NT, H), wmap2),
            pl.BlockSpec((1, H), wmap2),
            pl.BlockSpec((3 * H, H), wmap2),
            pl.BlockSpec((3 * H, H), wmap2),
            pl.BlockSpec((1, 3 * H), wmap2),
            pl.BlockSpec((1, 3 * H), wmap2),
            pl.BlockSpec((H, H), wmap2),
            pl.BlockSpec((1, H), wmap2),
            pl.BlockSpec((H, 1), wmap2),
            pl.BlockSpec((1, 1), wmap2),
        ],
        out_specs=pl.BlockSpec((_BB, NN - 1, 1), bmap),
        out_shape=jax.ShapeDtypeStruct((B, NN - 1, 1), jnp.float32),
    )(node_features, e_pad, rel3, hp3, t2, W_t, b_t2, W_msg, b_msg2,
      W_ih, W_hh, b_ih2, b_hh2, W_p1, b_p12, W_p2, b_p22)
    return out.reshape(B, NN - 1)


# speedup vs baseline: 1.3119x; 1.0017x over previous
import jax
import jax.numpy as jnp
from jax.experimental import pallas as pl
from jax.experimental.pallas import tpu as pltpu

_NN = 254
_NP = 256
_H = 128
_BB = 16


def _tree_kernel(x_ref, e_ref, rel_ref, hp_ref, t_ref,
                 Wt_ref, bt_ref, Wmsg_ref, bmsg_ref,
                 Wih_ref, Whh_ref, bih_ref, bhh_ref,
                 Wp1_ref, bp1_ref, Wp2_ref, bp2_ref,
                 out_ref):
    f32 = jnp.float32
    t = t_ref[0, 0]

    half = _H // 2
    kf = jax.lax.broadcasted_iota(jnp.int32, (1, half), 1).astype(f32)
    freqs = jnp.exp(-jnp.log(10000.0) * kf / half)
    ang = t * freqs
    emb = jnp.concatenate([jnp.sin(ang), jnp.cos(ang)], axis=1)
    temb = jnp.dot(emb, Wt_ref[...], preferred_element_type=f32) + bt_ref[...]

    Wtop = Wmsg_ref[:_H, :]
    Wbot = Wmsg_ref[_H:, :]
    Wd = Wtop - Wbot
    iota_l = jax.lax.broadcasted_iota(jnp.int32, (_NP, _NP), 1)
    iota_s = jax.lax.broadcasted_iota(jnp.int32, (_NP, _NN - 2), 0)

    zrows = jnp.zeros((_NP - _NN, _H), f32)
    x_all = jnp.concatenate(
        [p for b in range(_BB) for p in (x_ref[b], zrows)], axis=0)

    u_all = jnp.dot(x_all, Wd, preferred_element_type=f32) + bmsg_ref[...]
    v_all = jnp.dot(x_all, Wbot, preferred_element_type=f32)

    gnn_parts = []
    h0_parts = []
    ponehots = []
    for b in range(_BB):
        e = e_ref[b]
        u = u_all[b * _NP:(b + 1) * _NP]
        v = v_all[b * _NP:(b + 1) * _NP]
        gnn = jnp.zeros((_NP, _H), f32)
        for k in range(3):
            idx_col = e[:, k:k + 1]
            onehot = (iota_l == idx_col).astype(f32)
            if k == 0:
                ponehots.append(onehot)
            nv = jnp.dot(onehot, v, preferred_element_type=f32)
            gnn = gnn + jnp.maximum(u + nv, 0.0)
        gnn_parts.append(gnn)

        rel = rel_ref[b]
        scat = (iota_s == rel).astype(f32)
        h0_parts.append(jnp.dot(scat, hp_ref[b],
                                preferred_element_type=f32))

    gnn_all = jnp.concatenate(gnn_parts, axis=0)
    h0_all = jnp.concatenate(h0_parts, axis=0)

    gi = jax.lax.dot_general(gnn_all, Wih_ref[...], (((1,), (1,)), ((), ())),
                             preferred_element_type=f32) + bih_ref[...]
    gh = jax.lax.dot_general(h0_all, Whh_ref[...], (((1,), (1,)), ((), ())),
                             preferred_element_type=f32) + bhh_ref[...]
    r = jax.nn.sigmoid(gi[:, :_H] + gh[:, :_H])
    z = jax.nn.sigmoid(gi[:, _H:2 * _H] + gh[:, _H:2 * _H])
    n = jnp.tanh(gi[:, 2 * _H:] + r * gh[:, 2 * _H:])
    h_new = (1.0 - z) * n + z * h0_all

    ef_parts = []
    for b in range(_BB):
        hb = h_new[b * _NP:(b + 1) * _NP]
        pf = jnp.dot(ponehots[b], hb, preferred_element_type=f32)
        ef_parts.append(jnp.maximum(hb, pf))
    ef_all = jnp.concatenate(ef_parts, axis=0)

    ee = jnp.maximum(
        jnp.dot(ef_all, Wp1_ref[...], preferred_element_type=f32)
        + bp1_ref[...] + temb, 0.0)
    o = jnp.dot(ee, Wp2_ref[...], preferred_element_type=f32) + bp2_ref[0, 0]
    for b in range(_BB):
        out_ref[b] = o[b * _NP:b * _NP + (_NN - 1)]


def kernel(node_features, edge_index, rel_pos, h_prev, t, W_t, b_t, W_msg,
           b_msg, W_ih, W_hh, b_ih, b_hh, W_p1, b_p1, W_p2, b_p2):
    B, NN, NT = node_features.shape
    H = W_t.shape[0]
    NP = _NP

    e_pad = jnp.pad(edge_index, ((0, 0), (0, NP - NN), (0, 0)))
    rel3 = rel_pos.reshape(B, 1, NN - 2)
    hp3 = h_prev.reshape(B, NN - 2, H)
    t2 = t.reshape(1, 1)
    b_t2 = b_t.reshape(1, H)
    b_msg2 = b_msg.reshape(1, H)
    b_ih2 = b_ih.reshape(1, 3 * H)
    b_hh2 = b_hh.reshape(1, 3 * H)
    b_p12 = b_p1.reshape(1, H)
    b_p22 = b_p2.reshape(1, 1)

    def bmap(b):
        return (b, 0, 0)

    def wmap2(b):
        return (0, 0)

    out = pl.pallas_call(
        _tree_kernel,
        grid=(B // _BB,),
        in_specs=[
            pl.BlockSpec((_BB, NN, NT), bmap),
            pl.BlockSpec((_BB, NP, 3), bmap),
            pl.BlockSpec((_BB, 1, NN - 2), bmap),
            pl.BlockSpec((_BB, NN - 2, H), bmap),
            pl.BlockSpec((1, 1), wmap2),
            pl.BlockSpec((H, H), wmap2),
            pl.BlockSpec((1, H), wmap2),
            pl.BlockSpec((2 * NT, H), wmap2),
            pl.BlockSpec((1, H), wmap2),
            pl.BlockSpec((3 * H, H), wmap2),
            pl.BlockSpec((3 * H, H), wmap2),
            pl.BlockSpec((1, 3 * H), wmap2),
            pl.BlockSpec((1, 3 * H), wmap2),
            pl.BlockSpec((H, H), wmap2),
            pl.BlockSpec((1, H), wmap2),
            pl.BlockSpec((H, 1), wmap2),
            pl.BlockSpec((1, 1), wmap2),
        ],
        out_specs=pl.BlockSpec((_BB, NN - 1, 1), bmap),
        out_shape=jax.ShapeDtypeStruct((B, NN - 1, 1), jnp.float32),
        compiler_params=pltpu.CompilerParams(
            dimension_semantics=("parallel",)),
    )(node_features, e_pad, rel3, hp3, t2, W_t, b_t2, W_msg, b_msg2,
      W_ih, W_hh, b_ih2, b_hh2, W_p1, b_p12, W_p2, b_p22)
    return out.reshape(B, NN - 1)
